# baseline (speedup 1.0000x reference)
import functools

import jax
import jax.numpy as jnp
from jax import lax
from jax.experimental import pallas as pl
from jax.experimental.pallas import tpu as pltpu
from jax.experimental.pallas import tpu_sc as plsc

_SEQ = 2048
_DIM = 768
_SCALE = 8.0
_SPLIT = 1536

_info = plsc.get_sparse_core_info()
_NC, _NS = _info.num_cores, _info.num_subcores
_NW = _NC * _NS

_mesh = plsc.VectorSubcoreMesh(core_axis_name="c", subcore_axis_name="s")


def _make_gather(offset, seq):
    b_per_w = seq // _NW
    nb = max(1, b_per_w // 16)
    blk = b_per_w // nb

    @functools.partial(
        pl.kernel,
        mesh=_mesh,
        out_type=jax.ShapeDtypeStruct((seq, _DIM), jnp.float32),
        scratch_types=[
            pltpu.VMEM((b_per_w,), jnp.int32),
            pltpu.VMEM((b_per_w, _DIM), jnp.float32),
            pltpu.SemaphoreType.DMA((nb,)),
            pltpu.SemaphoreType.DMA,
        ],
    )
    def _emb_gather(table_hbm, ids_hbm, out_hbm, idx_v, rows_v, gsems, osem):
        wid = lax.axis_index("s") * _NC + lax.axis_index("c")
        base = wid * b_per_w
        pltpu.sync_copy(ids_hbm.at[pl.ds(offset + base, b_per_w)], idx_v)

        def fire(b, _):
            sl = pl.ds(b * blk, blk)
            pltpu.async_copy(table_hbm.at[idx_v.at[sl]], rows_v.at[sl], gsems.at[b])
            return _

        lax.fori_loop(0, nb, fire, None)

        def process(b, _):
            pltpu.make_async_copy(
                table_hbm.at[idx_v.at[pl.ds(b * blk, blk)]],
                rows_v.at[pl.ds(b * blk, blk)],
                gsems.at[b],
            ).wait()

            def scale_row(i, _):
                for j in range(_DIM // 16):
                    sl = pl.ds(j * 16, 16)
                    rows_v[i, sl] = rows_v[i, sl] * _SCALE
                return _

            lax.fori_loop(b * blk, (b + 1) * blk, scale_row, None)
            sl = pl.ds(b * blk, blk)
            pltpu.async_copy(
                rows_v.at[sl], out_hbm.at[pl.ds(base + b * blk, blk)], osem
            )
            return _

        lax.fori_loop(0, nb, process, None)

        pltpu.make_async_copy(rows_v, out_hbm.at[pl.ds(base, b_per_w)], osem).wait()

    return _emb_gather


_gather_a = _make_gather(0, _SPLIT)
_gather_b = _make_gather(_SPLIT, _SEQ - _SPLIT)


def kernel(token_ids, per_layer_table):
    b, s = token_ids.shape
    ids = token_ids.reshape(-1).astype(jnp.int32)
    r1 = _gather_a(per_layer_table, ids)
    r2 = _gather_b(per_layer_table, ids)
    out = jnp.zeros((b, s, 12, 64), jnp.float32)
    out = lax.dynamic_update_slice(out, r1.reshape(b, _SPLIT, 12, 64), (0, 0, 0, 0))
    out = lax.dynamic_update_slice(
        out, r2.reshape(b, s - _SPLIT, 12, 64), (0, _SPLIT, 0, 0)
    )
    return out


# speedup vs baseline: 1.0945x; 1.0945x over previous
import functools

import jax
import jax.numpy as jnp
from jax import lax
from jax.experimental import pallas as pl
from jax.experimental.pallas import tpu as pltpu
from jax.experimental.pallas import tpu_sc as plsc

_SEQ = 2048
_DIM = 768
_SCALE = 8.0

_info = plsc.get_sparse_core_info()
_NC, _NS = _info.num_cores, _info.num_subcores
_NW = _NC * _NS
_B_PER_W = _SEQ // _NW
_NB = 4
_BLK = _B_PER_W // _NB

_mesh = plsc.VectorSubcoreMesh(core_axis_name="c", subcore_axis_name="s")


@functools.partial(
    pl.kernel,
    mesh=_mesh,
    out_type=jax.ShapeDtypeStruct((_SEQ, _DIM), jnp.float32),
    scratch_types=[
        pltpu.VMEM((_B_PER_W,), jnp.int32),
        pltpu.VMEM((_B_PER_W, _DIM), jnp.float32),
        pltpu.SemaphoreType.DMA((_NB,)),
        pltpu.SemaphoreType.DMA,
    ],
)
def _emb_gather(table_hbm, ids_hbm, out_hbm, idx_v, rows_v, gsems, osem):
    wid = lax.axis_index("s") * _NC + lax.axis_index("c")
    base = wid * _B_PER_W
    pltpu.sync_copy(ids_hbm.at[pl.ds(base, _B_PER_W)], idx_v)

    def fire(b, _):
        blk = pl.ds(b * _BLK, _BLK)
        pltpu.async_copy(table_hbm.at[idx_v.at[blk]], rows_v.at[blk], gsems.at[b])
        return _

    lax.fori_loop(0, _NB, fire, None)

    def process(b, _):
        pltpu.make_async_copy(
            table_hbm.at[idx_v.at[pl.ds(b * _BLK, _BLK)]],
            rows_v.at[pl.ds(b * _BLK, _BLK)],
            gsems.at[b],
        ).wait()

        def scale_row(i, _):
            for j in range(_DIM // 16):
                sl = pl.ds(j * 16, 16)
                rows_v[i, sl] = rows_v[i, sl] * _SCALE
            return _

        lax.fori_loop(b * _BLK, (b + 1) * _BLK, scale_row, None)
        blk = pl.ds(b * _BLK, _BLK)
        pltpu.async_copy(
            rows_v.at[blk], out_hbm.at[pl.ds(base + b * _BLK, _BLK)], osem
        )
        return _

    lax.fori_loop(0, _NB, process, None)

    pltpu.make_async_copy(rows_v, out_hbm.at[pl.ds(base, _B_PER_W)], osem).wait()


def kernel(token_ids, per_layer_table):
    b, s = token_ids.shape
    ids = token_ids.reshape(-1).astype(jnp.int32)
    out = _emb_gather(per_layer_table, ids)
    return out.reshape(b, s, 12, 64)
